# Initial kernel scaffold; baseline (speedup 1.0000x reference)
import jax
import jax.numpy as jnp
from jax.experimental import pallas as pl


def kernel(x, edge_list, edge_feature, edge_weight, W_edge, b_edge):
    raise NotImplementedError("write your pallas kernel here")


# speedup vs baseline: 1.3999x; 1.3999x over previous
import functools

import jax
import jax.numpy as jnp
from jax import lax
from jax.experimental import pallas as pl
from jax.experimental.pallas import tpu as pltpu
from jax.experimental.pallas import tpu_sc as plsc

N_NODES = 10000
D = 16
WINDOW = 128
NUM_TILES = 32
BE = 2048

@functools.cache
def _mesh():
    return plsc.VectorSubcoreMesh(core_axis_name="core", subcore_axis_name="subcore")


_SC_PARAMS = pltpu.CompilerParams(use_tc_tiling_on_sc=False)


def _gather_src(x, idx_row, e_pad):
    @functools.partial(
        pl.kernel,
        out_type=jax.ShapeDtypeStruct((e_pad, D), jnp.float32),
        mesh=_mesh(),
        compiler_params=_SC_PARAMS,
    )
    def k(x_hbm, i_hbm, o_hbm):
        def body(i_vmem, o_vmem):
            pltpu.sync_copy(x_hbm.at[i_vmem.at[0]], o_vmem)

        pltpu.emit_pipeline(
            body,
            grid=(e_pad // WINDOW,),
            in_specs=[pl.BlockSpec((1, WINDOW), lambda i: (0, i))],
            out_specs=[pl.BlockSpec((WINDOW, D), lambda i: (i, 0))],
            core_axis_name=("core", "subcore"),
            dimension_semantics=(pltpu.PARALLEL,),
        )(i_hbm, o_hbm)

    return k(x, idx_row)


def _dense_messages(ef, src, w, vfull, e_pad):
    def body(ef_ref, src_ref, w_ref, v_ref, o_ref):
        srcw = src_ref[...] * w_ref[...]
        u = jnp.dot(srcw, v_ref[...], preferred_element_type=jnp.float32,
                    precision=lax.Precision.HIGHEST)
        msg = u[:, D * D:D * D + D]
        for k in range(D):
            msg = msg + ef_ref[:, k:k + 1] * u[:, k * D:(k + 1) * D]
        o_ref[...] = msg

    grid = (e_pad // BE,)
    return pl.pallas_call(
        body,
        grid=grid,
        in_specs=[
            pl.BlockSpec((BE, D), lambda i: (i, 0)),
            pl.BlockSpec((BE, D), lambda i: (i, 0)),
            pl.BlockSpec((BE, 1), lambda i: (i, 0)),
            pl.BlockSpec((D, D * D + D), lambda i: (0, 0)),
        ],
        out_specs=pl.BlockSpec((BE, D), lambda i: (i, 0)),
        out_shape=jax.ShapeDtypeStruct((e_pad, D), jnp.float32),
    )(ef, src, w, vfull)


def _scatter_add(msg, idx_row, zeros_nd, e_pad):
    @functools.partial(
        pl.kernel,
        out_type=jax.ShapeDtypeStruct((2, N_NODES, D), jnp.float32),
        mesh=_mesh(),
        scratch_types=[pltpu.VMEM_SHARED((N_NODES, D), jnp.float32)],
        compiler_params=_SC_PARAMS,
    )
    def k(m_hbm, i_hbm, z_hbm, o_hbm, acc):
        c = lax.axis_index("core")
        s = lax.axis_index("subcore")

        @pl.when(s == 0)
        def _():
            pltpu.sync_copy(z_hbm, acc)

        plsc.subcore_barrier()

        def body(m_vmem, i_vmem):
            pltpu.sync_copy(m_vmem, acc.at[i_vmem.at[0]], add=True)

        pltpu.emit_pipeline(
            body,
            grid=(e_pad // WINDOW,),
            in_specs=[
                pl.BlockSpec((WINDOW, D), lambda i: (i, 0)),
                pl.BlockSpec((1, WINDOW), lambda i: (0, i)),
            ],
            out_specs=[],
            core_axis_name=("core", "subcore"),
            dimension_semantics=(pltpu.PARALLEL,),
        )(m_hbm, i_hbm)

        plsc.subcore_barrier()
        rows = N_NODES // 16
        pltpu.sync_copy(acc.at[pl.ds(s * rows, rows)],
                        o_hbm.at[c].at[pl.ds(s * rows, rows)])

    return k(msg, idx_row, zeros_nd)


def _combine_relu(partials):
    def body(p_ref, o_ref):
        o_ref[...] = jnp.maximum(p_ref[0] + p_ref[1], 0.0)

    return pl.pallas_call(
        body,
        out_shape=jax.ShapeDtypeStruct((N_NODES, D), jnp.float32),
    )(partials)


def kernel(x, edge_list, edge_feature, edge_weight, W_edge, b_edge):
    e = edge_list.shape[0]
    e_pad = ((e + NUM_TILES * WINDOW - 1) // (NUM_TILES * WINDOW)) * (NUM_TILES * WINDOW)
    pad = e_pad - e

    node_in = jnp.pad(edge_list[:, 0].astype(jnp.int32), (0, pad))
    node_out = jnp.pad(edge_list[:, 1].astype(jnp.int32), (0, pad))
    ef = jnp.pad(edge_feature.astype(jnp.float32), ((0, pad), (0, 0)))
    w = jnp.pad(edge_weight.astype(jnp.float32), (0, pad))[:, None]

    w3 = W_edge.astype(jnp.float32).reshape(D, D, D)
    va = jnp.transpose(w3, (2, 0, 1)).reshape(D, D * D)
    bmat_t = b_edge.astype(jnp.float32).reshape(D, D).T
    vfull = jnp.concatenate([va, bmat_t], axis=1)

    src = _gather_src(x.astype(jnp.float32), node_in[None, :], e_pad)
    msg = _dense_messages(ef, src, w, vfull, e_pad)
    partials = _scatter_add(msg, node_out[None, :], jnp.zeros((N_NODES, D), jnp.float32), e_pad)
    return _combine_relu(partials)
